# baseline (speedup 1.0000x reference)
import functools

import jax
import jax.numpy as jnp
from jax.experimental import pallas as pl


_D = 256
_L = 7
_MAXP = 32
_BB = 32
_NCH = 8
_NEG = -1e30


def _topk_mask(s, k):
    bits = jax.lax.bitcast_convert_type(s, jnp.int32)
    key = jnp.where(bits < 0, bits ^ jnp.int32(0x7FFFFFFF), bits)
    kf = float(k)
    cnt = jnp.sum((key >= 0).astype(jnp.float32), axis=1, keepdims=True)
    base = jnp.where(cnt >= kf,
                     jnp.zeros_like(key[:, :1]),
                     jnp.full_like(key[:, :1], jnp.int32(-2**31)))
    for j in range(30, -1, -1):
        cand = base | jnp.int32(1 << j)
        cnt = jnp.sum((key >= cand).astype(jnp.float32), axis=1,
                      keepdims=True)
        base = jnp.where(cnt >= kf, cand, base)
    return key >= base


def _body(x_ref, wcat_ref, bias_ref, wq1_ref, bq1_ref, wq2_ref, bq2_ref,
          out_ref):
    f32 = jnp.float32
    bf16 = jnp.bfloat16

    def dot16(a, b):
        return jnp.dot(a.astype(bf16), b.astype(bf16),
                       preferred_element_type=f32)

    def score(tokens, j, paths):
        h = dot16(tokens, wq1_ref[j])
        h = jnp.maximum(h + bq1_ref[j], 0.0)
        s8 = dot16(h, wq2_ref[j])
        s = jnp.sum(s8.reshape(paths, _BB, 8), axis=2)
        return s + bq2_ref[j]

    def chain(x):
        mask64 = None
        for i in range(_L):
            y = dot16(x, wcat_ref[i])
            y = y + bias_ref[i]
            x = jnp.concatenate([y[:, :_D], y[:, _D:]], axis=0)
            if i == _L - 2:
                s5 = score(x, 0, 64).T
                mask64 = _topk_mask(s5, _MAXP)
            if i < _L - 1:
                x = jnp.tanh(x)

        s6 = score(x, 1, 128).T
        s6 = jnp.concatenate([jnp.where(mask64, s6[:, :64], _NEG),
                              jnp.where(mask64, s6[:, 64:], _NEG)], axis=1)
        keep = _topk_mask(s6, _MAXP)
        s6 = jnp.where(keep, s6, _NEG)
        m = jnp.max(s6, axis=1, keepdims=True)
        e = jnp.exp(s6 - m)
        wt = e / jnp.sum(e, axis=1, keepdims=True)
        acc = jnp.zeros((_BB, _D), f32)
        for p in range(128):
            acc = acc + x[p * _BB:(p + 1) * _BB, :] * wt[:, p:p + 1]
        return acc

    for c in range(_NCH):
        out_ref[c * _BB:(c + 1) * _BB, :] = chain(
            x_ref[c * _BB:(c + 1) * _BB, :])


@jax.jit
def kernel(x, Wm, bm, Ws, Wq1, bq1, Wq2, bq2):
    batch, d = x.shape
    num_layers = Wm.shape[0]

    wcat = jnp.concatenate(
        [jnp.swapaxes(Wm, 1, 2), jnp.swapaxes(Ws, 1, 2)],
        axis=2).astype(jnp.bfloat16)
    bias = jnp.concatenate([bm, jnp.zeros_like(bm)], axis=1)[:, None, :]
    wq1t = jnp.swapaxes(Wq1[num_layers - 2:], 1, 2).astype(jnp.bfloat16)
    bq1s = bq1[num_layers - 2:][:, None, :]
    wq2s = jnp.zeros((2, 32, 8), jnp.float32).at[:, :, 0].set(
        Wq2[num_layers - 2:, 0, :]).astype(jnp.bfloat16)
    bq2s = jnp.broadcast_to(bq2[num_layers - 2:][:, :, None],
                            (2, 1, _BB)) * jnp.ones((2, 1, _BB))

    blk = _BB * _NCH
    grid = (batch // blk,)
    full = lambda *shape: pl.BlockSpec(shape, lambda i: (0,) * len(shape))
    return pl.pallas_call(
        _body,
        grid=grid,
        in_specs=[
            pl.BlockSpec((blk, d), lambda i: (i, 0)),
            full(num_layers, d, 2 * d),
            full(num_layers, 1, 2 * d),
            full(2, d, 32),
            full(2, 1, 32),
            full(2, 32, 8),
            full(2, 1, _BB),
        ],
        out_specs=pl.BlockSpec((blk, d), lambda i: (i, 0)),
        out_shape=jax.ShapeDtypeStruct((batch, d), jnp.float32),
    )(x, wcat, bias, wq1t, bq1s, wq2s, bq2s)


# speedup vs baseline: 1.1477x; 1.1477x over previous
import functools

import jax
import jax.numpy as jnp
from jax.experimental import pallas as pl


_D = 256
_L = 7
_MAXP = 32
_BB = 32
_NCH = 8
_NEG = -1e30


def _topk_mask(s, k):
    bits = jax.lax.bitcast_convert_type(s, jnp.int32)
    key = jnp.where(bits < 0, bits ^ jnp.int32(0x7FFFFFFF), bits)
    kf = float(k)
    cnt = jnp.sum((key >= 0).astype(jnp.float32), axis=1, keepdims=True)
    base = jnp.where(cnt >= kf,
                     jnp.zeros_like(key[:, :1]),
                     jnp.full_like(key[:, :1], jnp.int32(-2**31)))
    for j in range(30, -1, -1):
        cand = base | jnp.int32(1 << j)
        cnt = jnp.sum((key >= cand).astype(jnp.float32), axis=1,
                      keepdims=True)
        base = jnp.where(cnt >= kf, cand, base)
    return key >= base


def _body(x_ref, wcat_ref, bias_ref, wq1_ref, bq1_ref, wq2_ref, bq2_ref,
          out_ref):
    f32 = jnp.float32
    bf16 = jnp.bfloat16

    def dot16(a, b):
        return jnp.dot(a.astype(bf16), b.astype(bf16),
                       preferred_element_type=f32)

    def score(tokens, j, paths):
        h = dot16(tokens, wq1_ref[j])
        h = jnp.maximum(h + bq1_ref[j], 0.0)
        h3 = h.reshape(paths, _BB, 32).astype(bf16).astype(f32)
        w2 = wq2_ref[j][None].astype(f32)
        s = jnp.sum(h3 * w2, axis=2)
        return s + bq2_ref[j]

    def chain(x):
        mask64 = None
        for i in range(_L):
            y = dot16(x, wcat_ref[i])
            y = y + bias_ref[i]
            x = jnp.concatenate([y[:, :_D], y[:, _D:]], axis=0)
            if i == _L - 2:
                s5 = score(x, 0, 64).T
                mask64 = _topk_mask(s5, _MAXP)
            if i < _L - 1:
                x = jnp.tanh(x)

        s6 = score(x, 1, 128).T
        s6 = jnp.concatenate([jnp.where(mask64, s6[:, :64], _NEG),
                              jnp.where(mask64, s6[:, 64:], _NEG)], axis=1)
        keep = _topk_mask(s6, _MAXP)
        s6 = jnp.where(keep, s6, _NEG)
        m = jnp.max(s6, axis=1, keepdims=True)
        e = jnp.exp(s6 - m)
        wt = e / jnp.sum(e, axis=1, keepdims=True)
        acc = jnp.zeros((_BB, _D), f32)
        for p in range(128):
            acc = acc + x[p * _BB:(p + 1) * _BB, :] * wt[:, p:p + 1]
        return acc

    for c in range(_NCH):
        out_ref[c * _BB:(c + 1) * _BB, :] = chain(
            x_ref[c * _BB:(c + 1) * _BB, :])


@jax.jit
def kernel(x, Wm, bm, Ws, Wq1, bq1, Wq2, bq2):
    batch, d = x.shape
    num_layers = Wm.shape[0]

    wcat = jnp.concatenate(
        [jnp.swapaxes(Wm, 1, 2), jnp.swapaxes(Ws, 1, 2)],
        axis=2).astype(jnp.bfloat16)
    bias = jnp.concatenate([bm, jnp.zeros_like(bm)], axis=1)[:, None, :]
    wq1t = jnp.swapaxes(Wq1[num_layers - 2:], 1, 2).astype(jnp.bfloat16)
    bq1s = bq1[num_layers - 2:][:, None, :]
    wq2s = Wq2[num_layers - 2:, 0, :][:, None, :].astype(jnp.bfloat16)
    bq2s = jnp.broadcast_to(bq2[num_layers - 2:][:, :, None],
                            (2, 1, _BB)) * jnp.ones((2, 1, _BB))

    blk = _BB * _NCH
    grid = (batch // blk,)
    full = lambda *shape: pl.BlockSpec(shape, lambda i: (0,) * len(shape))
    return pl.pallas_call(
        _body,
        grid=grid,
        in_specs=[
            pl.BlockSpec((blk, d), lambda i: (i, 0)),
            full(num_layers, d, 2 * d),
            full(num_layers, 1, 2 * d),
            full(2, d, 32),
            full(2, 1, 32),
            full(2, 1, 32),
            full(2, 1, _BB),
        ],
        out_specs=pl.BlockSpec((blk, d), lambda i: (i, 0)),
        out_shape=jax.ShapeDtypeStruct((batch, d), jnp.float32),
    )(x, wcat, bias, wq1t, bq1s, wq2s, bq2s)


# speedup vs baseline: 3.1716x; 2.7635x over previous
import functools

import jax
import jax.numpy as jnp
from jax.experimental import pallas as pl


_D = 256
_L = 7
_MAXP = 32
_BB = 32
_NCH = 8
_NEG = -1e30


def _topk_mask(s, k):
    bits = jax.lax.bitcast_convert_type(s, jnp.int32)
    key = jnp.where(bits < 0, bits ^ jnp.int32(0x7FFFFFFF), bits)
    kf = float(k)
    cnt = jnp.sum((key >= 0).astype(jnp.float32), axis=1, keepdims=True)
    base = jnp.where(cnt >= kf,
                     jnp.zeros_like(key[:, :1]),
                     jnp.full_like(key[:, :1], jnp.int32(-2**31)))
    for j in range(30, -1, -1):
        cand = base | jnp.int32(1 << j)
        cnt = jnp.sum((key >= cand).astype(jnp.float32), axis=1,
                      keepdims=True)
        base = jnp.where(cnt >= kf, cand, base)
    return key >= base


def _body(x_ref, wcat_ref, bias_ref, wq1_ref, bq1_ref, wq2_ref, bq2_ref,
          out_ref):
    f32 = jnp.float32
    bf16 = jnp.bfloat16

    def dot16(a, b):
        return jnp.dot(a.astype(bf16), b.astype(bf16),
                       preferred_element_type=f32)

    def score(tokens, j, paths):
        h = dot16(tokens, wq1_ref[j])
        h = jnp.maximum(h + bq1_ref[j], 0.0)
        h3 = h.reshape(paths, _BB, 32).astype(bf16).astype(f32)
        w2 = wq2_ref[j][None].astype(bf16).astype(f32)
        s = jnp.sum(h3 * w2, axis=2)
        return s + bq2_ref[j]

    def chain(x):
        mask64 = None
        for i in range(_L):
            y = dot16(x, wcat_ref[i])
            y = y + bias_ref[i]
            x = jnp.concatenate([y[:, :_D], y[:, _D:]], axis=0)
            if i == _L - 2:
                s5 = score(x, 0, 64).T
                mask64 = _topk_mask(s5, _MAXP)
            if i < _L - 1:
                x = jnp.tanh(x)

        s6 = score(x, 1, 128).T
        s6 = jnp.concatenate([jnp.where(mask64, s6[:, :64], _NEG),
                              jnp.where(mask64, s6[:, 64:], _NEG)], axis=1)
        keep = _topk_mask(s6, _MAXP)
        s6 = jnp.where(keep, s6, _NEG)
        m = jnp.max(s6, axis=1, keepdims=True)
        e = jnp.exp(s6 - m)
        wt = e / jnp.sum(e, axis=1, keepdims=True)
        acc = jnp.zeros((_BB, _D), f32)
        for p in range(128):
            acc = acc + x[p * _BB:(p + 1) * _BB, :] * wt[:, p:p + 1]
        return acc

    for c in range(_NCH):
        out_ref[c * _BB:(c + 1) * _BB, :] = chain(
            x_ref[c * _BB:(c + 1) * _BB, :])


@jax.jit
def kernel(x, Wm, bm, Ws, Wq1, bq1, Wq2, bq2):
    batch, d = x.shape
    num_layers = Wm.shape[0]

    wcat = jnp.concatenate(
        [jnp.swapaxes(Wm, 1, 2), jnp.swapaxes(Ws, 1, 2)],
        axis=2).astype(jnp.bfloat16)
    bias = jnp.concatenate([bm, jnp.zeros_like(bm)], axis=1)[:, None, :]
    wq1t = jnp.swapaxes(Wq1[num_layers - 2:], 1, 2).astype(jnp.bfloat16)
    bq1s = bq1[num_layers - 2:][:, None, :]
    wq2s = Wq2[num_layers - 2:, 0, :][:, None, :]
    bq2s = bq2[num_layers - 2:][:, :, None]

    blk = _BB * _NCH
    grid = (batch // blk,)
    full = lambda *shape: pl.BlockSpec(shape, lambda i: (0,) * len(shape))
    return pl.pallas_call(
        _body,
        grid=grid,
        in_specs=[
            pl.BlockSpec((blk, d), lambda i: (i, 0)),
            full(num_layers, d, 2 * d),
            full(num_layers, 1, 2 * d),
            full(2, d, 32),
            full(2, 1, 32),
            full(2, 1, 32),
            full(2, 1, 1),
        ],
        out_specs=pl.BlockSpec((blk, d), lambda i: (i, 0)),
        out_shape=jax.ShapeDtypeStruct((batch, d), jnp.float32),
    )(x, wcat, bias, wq1t, bq1s, wq2s, bq2s)


# speedup vs baseline: 3.3783x; 1.0652x over previous
import functools

import jax
import jax.numpy as jnp
from jax.experimental import pallas as pl


_D = 256
_L = 7
_MAXP = 32
_BB = 64
_NCH = 4
_NEG = -1e30


def _topk_mask(s, k):
    bits = jax.lax.bitcast_convert_type(s, jnp.int32)
    key = jnp.where(bits < 0, bits ^ jnp.int32(0x7FFFFFFF), bits)
    kf = float(k)
    cnt = jnp.sum((key >= 0).astype(jnp.float32), axis=1, keepdims=True)
    base = jnp.where(cnt >= kf,
                     jnp.zeros_like(key[:, :1]),
                     jnp.full_like(key[:, :1], jnp.int32(-2**31)))
    for j in range(30, -1, -1):
        cand = base | jnp.int32(1 << j)
        cnt = jnp.sum((key >= cand).astype(jnp.float32), axis=1,
                      keepdims=True)
        base = jnp.where(cnt >= kf, cand, base)
    return key >= base


def _body(x_ref, wcat_ref, bias_ref, wq1_ref, bq1_ref, wq2_ref, bq2_ref,
          out_ref):
    f32 = jnp.float32
    bf16 = jnp.bfloat16

    def dot16(a, b):
        return jnp.dot(a.astype(bf16), b.astype(bf16),
                       preferred_element_type=f32)

    def score(tokens, j, paths):
        h = dot16(tokens, wq1_ref[j])
        h = jnp.maximum(h + bq1_ref[j], 0.0)
        h3 = h.reshape(paths, _BB, 32).astype(bf16).astype(f32)
        w2 = wq2_ref[j][None].astype(bf16).astype(f32)
        s = jnp.sum(h3 * w2, axis=2)
        return s + bq2_ref[j]

    def chain(x):
        mask64 = None
        for i in range(_L):
            y = dot16(x, wcat_ref[i])
            y = y + bias_ref[i]
            x = jnp.concatenate([y[:, :_D], y[:, _D:]], axis=0)
            if i == _L - 2:
                s5 = score(x, 0, 64).T
                mask64 = _topk_mask(s5, _MAXP)
            if i < _L - 1:
                x = jnp.tanh(x)

        s6 = score(x, 1, 128).T
        s6 = jnp.concatenate([jnp.where(mask64, s6[:, :64], _NEG),
                              jnp.where(mask64, s6[:, 64:], _NEG)], axis=1)
        keep = _topk_mask(s6, _MAXP)
        s6 = jnp.where(keep, s6, _NEG)
        m = jnp.max(s6, axis=1, keepdims=True)
        e = jnp.exp(s6 - m)
        wt = e / jnp.sum(e, axis=1, keepdims=True)
        acc = jnp.zeros((_BB, _D), f32)
        for p in range(128):
            acc = acc + x[p * _BB:(p + 1) * _BB, :] * wt[:, p:p + 1]
        return acc

    for c in range(_NCH):
        out_ref[c * _BB:(c + 1) * _BB, :] = chain(
            x_ref[c * _BB:(c + 1) * _BB, :])


@jax.jit
def kernel(x, Wm, bm, Ws, Wq1, bq1, Wq2, bq2):
    batch, d = x.shape
    num_layers = Wm.shape[0]

    wcat = jnp.concatenate(
        [jnp.swapaxes(Wm, 1, 2), jnp.swapaxes(Ws, 1, 2)],
        axis=2).astype(jnp.bfloat16)
    bias = jnp.concatenate([bm, jnp.zeros_like(bm)], axis=1)[:, None, :]
    wq1t = jnp.swapaxes(Wq1[num_layers - 2:], 1, 2).astype(jnp.bfloat16)
    bq1s = bq1[num_layers - 2:][:, None, :]
    wq2s = Wq2[num_layers - 2:, 0, :][:, None, :]
    bq2s = bq2[num_layers - 2:][:, :, None]

    blk = _BB * _NCH
    grid = (batch // blk,)
    full = lambda *shape: pl.BlockSpec(shape, lambda i: (0,) * len(shape))
    return pl.pallas_call(
        _body,
        grid=grid,
        in_specs=[
            pl.BlockSpec((blk, d), lambda i: (i, 0)),
            full(num_layers, d, 2 * d),
            full(num_layers, 1, 2 * d),
            full(2, d, 32),
            full(2, 1, 32),
            full(2, 1, 32),
            full(2, 1, 1),
        ],
        out_specs=pl.BlockSpec((blk, d), lambda i: (i, 0)),
        out_shape=jax.ShapeDtypeStruct((batch, d), jnp.float32),
    )(x, wcat, bias, wq1t, bq1s, wq2s, bq2s)


# speedup vs baseline: 3.8409x; 1.1369x over previous
import functools

import jax
import jax.numpy as jnp
from jax.experimental import pallas as pl


_D = 256
_L = 7
_MAXP = 32
_BB = 128
_NCH = 2
_NEG = -1e30


def _topk_mask(s, k):
    bits = jax.lax.bitcast_convert_type(s, jnp.int32)
    key = jnp.where(bits < 0, bits ^ jnp.int32(0x7FFFFFFF), bits)
    kf = float(k)
    cnt = jnp.sum((key >= 0).astype(jnp.float32), axis=1, keepdims=True)
    base = jnp.where(cnt >= kf,
                     jnp.zeros_like(key[:, :1]),
                     jnp.full_like(key[:, :1], jnp.int32(-2**31)))
    for j in range(30, -1, -1):
        cand = base | jnp.int32(1 << j)
        cnt = jnp.sum((key >= cand).astype(jnp.float32), axis=1,
                      keepdims=True)
        base = jnp.where(cnt >= kf, cand, base)
    return key >= base


def _body(x_ref, wcat_ref, bias_ref, wq1_ref, bq1_ref, wq2_ref, bq2_ref,
          out_ref):
    f32 = jnp.float32
    bf16 = jnp.bfloat16

    def dot16(a, b):
        return jnp.dot(a.astype(bf16), b.astype(bf16),
                       preferred_element_type=f32)

    def score(tokens, j, paths):
        h = dot16(tokens, wq1_ref[j])
        h = jnp.maximum(h + bq1_ref[j], 0.0)
        h3 = h.reshape(paths, _BB, 32).astype(bf16).astype(f32)
        w2 = wq2_ref[j][None].astype(bf16).astype(f32)
        s = jnp.sum(h3 * w2, axis=2)
        return s + bq2_ref[j]

    def chain(x):
        mask64 = None
        for i in range(_L):
            y = dot16(x, wcat_ref[i])
            y = y + bias_ref[i]
            x = jnp.concatenate([y[:, :_D], y[:, _D:]], axis=0)
            if i == _L - 2:
                s5 = score(x, 0, 64).T
                mask64 = _topk_mask(s5, _MAXP)
            if i < _L - 1:
                x = jnp.tanh(x)

        s6 = score(x, 1, 128).T
        s6 = jnp.concatenate([jnp.where(mask64, s6[:, :64], _NEG),
                              jnp.where(mask64, s6[:, 64:], _NEG)], axis=1)
        keep = _topk_mask(s6, _MAXP)
        s6 = jnp.where(keep, s6, _NEG)
        m = jnp.max(s6, axis=1, keepdims=True)
        e = jnp.exp(s6 - m)
        wt = e / jnp.sum(e, axis=1, keepdims=True)
        acc = jnp.zeros((_BB, _D), f32)
        for p in range(128):
            acc = acc + x[p * _BB:(p + 1) * _BB, :] * wt[:, p:p + 1]
        return acc

    for c in range(_NCH):
        out_ref[c * _BB:(c + 1) * _BB, :] = chain(
            x_ref[c * _BB:(c + 1) * _BB, :])


@jax.jit
def kernel(x, Wm, bm, Ws, Wq1, bq1, Wq2, bq2):
    batch, d = x.shape
    num_layers = Wm.shape[0]

    wcat = jnp.concatenate(
        [jnp.swapaxes(Wm, 1, 2), jnp.swapaxes(Ws, 1, 2)],
        axis=2).astype(jnp.bfloat16)
    bias = jnp.concatenate([bm, jnp.zeros_like(bm)], axis=1)[:, None, :]
    wq1t = jnp.swapaxes(Wq1[num_layers - 2:], 1, 2).astype(jnp.bfloat16)
    bq1s = bq1[num_layers - 2:][:, None, :]
    wq2s = Wq2[num_layers - 2:, 0, :][:, None, :]
    bq2s = bq2[num_layers - 2:][:, :, None]

    blk = _BB * _NCH
    grid = (batch // blk,)
    full = lambda *shape: pl.BlockSpec(shape, lambda i: (0,) * len(shape))
    return pl.pallas_call(
        _body,
        grid=grid,
        in_specs=[
            pl.BlockSpec((blk, d), lambda i: (i, 0)),
            full(num_layers, d, 2 * d),
            full(num_layers, 1, 2 * d),
            full(2, d, 32),
            full(2, 1, 32),
            full(2, 1, 32),
            full(2, 1, 1),
        ],
        out_specs=pl.BlockSpec((blk, d), lambda i: (i, 0)),
        out_shape=jax.ShapeDtypeStruct((batch, d), jnp.float32),
    )(x, wcat, bias, wq1t, bq1s, wq2s, bq2s)


# speedup vs baseline: 3.9332x; 1.0240x over previous
import functools

import jax
import jax.numpy as jnp
from jax.experimental import pallas as pl


_D = 256
_L = 7
_MAXP = 32
_BB = 256
_NCH = 1
_NEG = -1e30


def _topk_mask(s, k):
    bits = jax.lax.bitcast_convert_type(s, jnp.int32)
    key = jnp.where(bits < 0, bits ^ jnp.int32(0x7FFFFFFF), bits)
    kf = float(k)
    cnt = jnp.sum((key >= 0).astype(jnp.float32), axis=1, keepdims=True)
    base = jnp.where(cnt >= kf,
                     jnp.zeros_like(key[:, :1]),
                     jnp.full_like(key[:, :1], jnp.int32(-2**31)))
    for j in range(30, -1, -1):
        cand = base | jnp.int32(1 << j)
        cnt = jnp.sum((key >= cand).astype(jnp.float32), axis=1,
                      keepdims=True)
        base = jnp.where(cnt >= kf, cand, base)
    return key >= base


def _body(x_ref, wcat_ref, bias_ref, wq1_ref, bq1_ref, wq2_ref, bq2_ref,
          out_ref):
    f32 = jnp.float32
    bf16 = jnp.bfloat16

    def dot16(a, b):
        return jnp.dot(a.astype(bf16), b.astype(bf16),
                       preferred_element_type=f32)

    def score(tokens, j, paths):
        h = dot16(tokens, wq1_ref[j])
        h = jnp.maximum(h + bq1_ref[j], 0.0)
        h3 = h.reshape(paths, _BB, 32).astype(bf16).astype(f32)
        w2 = wq2_ref[j][None].astype(bf16).astype(f32)
        s = jnp.sum(h3 * w2, axis=2)
        return s + bq2_ref[j]

    def chain(x):
        mask64 = None
        for i in range(_L):
            y = dot16(x, wcat_ref[i])
            y = y + bias_ref[i]
            x = jnp.concatenate([y[:, :_D], y[:, _D:]], axis=0)
            if i == _L - 2:
                s5 = score(x, 0, 64).T
                mask64 = _topk_mask(s5, _MAXP)
            if i < _L - 1:
                x = jnp.tanh(x)

        s6 = score(x, 1, 128).T
        s6 = jnp.concatenate([jnp.where(mask64, s6[:, :64], _NEG),
                              jnp.where(mask64, s6[:, 64:], _NEG)], axis=1)
        keep = _topk_mask(s6, _MAXP)
        s6 = jnp.where(keep, s6, _NEG)
        m = jnp.max(s6, axis=1, keepdims=True)
        e = jnp.exp(s6 - m)
        wt = e / jnp.sum(e, axis=1, keepdims=True)
        acc = jnp.zeros((_BB, _D), f32)
        for p in range(128):
            acc = acc + x[p * _BB:(p + 1) * _BB, :] * wt[:, p:p + 1]
        return acc

    for c in range(_NCH):
        out_ref[c * _BB:(c + 1) * _BB, :] = chain(
            x_ref[c * _BB:(c + 1) * _BB, :])


@jax.jit
def kernel(x, Wm, bm, Ws, Wq1, bq1, Wq2, bq2):
    batch, d = x.shape
    num_layers = Wm.shape[0]

    wcat = jnp.concatenate(
        [jnp.swapaxes(Wm, 1, 2), jnp.swapaxes(Ws, 1, 2)],
        axis=2).astype(jnp.bfloat16)
    bias = jnp.concatenate([bm, jnp.zeros_like(bm)], axis=1)[:, None, :]
    wq1t = jnp.swapaxes(Wq1[num_layers - 2:], 1, 2).astype(jnp.bfloat16)
    bq1s = bq1[num_layers - 2:][:, None, :]
    wq2s = Wq2[num_layers - 2:, 0, :][:, None, :]
    bq2s = bq2[num_layers - 2:][:, :, None]

    blk = _BB * _NCH
    grid = (batch // blk,)
    full = lambda *shape: pl.BlockSpec(shape, lambda i: (0,) * len(shape))
    return pl.pallas_call(
        _body,
        grid=grid,
        in_specs=[
            pl.BlockSpec((blk, d), lambda i: (i, 0)),
            full(num_layers, d, 2 * d),
            full(num_layers, 1, 2 * d),
            full(2, d, 32),
            full(2, 1, 32),
            full(2, 1, 32),
            full(2, 1, 1),
        ],
        out_specs=pl.BlockSpec((blk, d), lambda i: (i, 0)),
        out_shape=jax.ShapeDtypeStruct((batch, d), jnp.float32),
    )(x, wcat, bias, wq1t, bq1s, wq2s, bq2s)
